# baseline (speedup 1.0000x reference)
import jax
import jax.numpy as jnp
from jax.experimental import pallas as pl
from jax.experimental.pallas import tpu as pltpu

_NE = 64
_K = 8
_B = 4096
_BB = 1024
_D = 128
_R = 32
_GRID = _B // _BB


def _gate_body(x_ref, w1_ref, b1_ref, w2_ref, b2_ref, w3_ref, b3_ref,
               probs_ref, tkp_ref, tki_ref, psum_ref):
    xb = x_ref[...]
    y = jnp.sum(xb, axis=1) * (1.0 / _R)
    ht = jnp.transpose(y)
    h = jnp.maximum(jnp.dot(w1_ref[...], ht, preferred_element_type=jnp.float32)
                    + b1_ref[...], 0.0)
    h = jnp.maximum(jnp.dot(w2_ref[...], h, preferred_element_type=jnp.float32)
                    + b2_ref[...], 0.0)
    logits = (jnp.dot(w3_ref[...], h, preferred_element_type=jnp.float32)
              + b3_ref[...])

    m = jnp.max(logits, axis=0, keepdims=True)
    e = jnp.exp(logits - m)
    p = e / jnp.sum(e, axis=0, keepdims=True)
    probs_ref[...] = p
    psum_ref[...] = jnp.sum(p, axis=1, keepdims=True)[None]

    idx = jax.lax.broadcasted_iota(jnp.int32, p.shape, 0).astype(jnp.float32)
    pc = p
    for k in range(_K):
        mv = jnp.max(pc, axis=0, keepdims=True)
        sel = jnp.min(jnp.where(pc == mv, idx, float(_NE)),
                      axis=0, keepdims=True)
        tkp_ref[pl.ds(k, 1), :] = mv
        tki_ref[pl.ds(k, 1), :] = sel.astype(jnp.int32)
        pc = jnp.where(idx == sel, -1.0, pc)


def _loss_body(psum_ref, loss_ref):
    acc = jnp.sum(psum_ref[...], axis=0)
    d = acc * (1.0 / _B) - (1.0 / _NE)
    loss_ref[...] = jnp.sum(d * d, axis=0, keepdims=True)


def kernel(x, W1, b1, W2, b2, W3, b3):
    xt = jnp.transpose(x, (0, 1, 3, 4, 2)).reshape(_B, _R, _D)

    probs_t, tkp_t, tki_t, psum = pl.pallas_call(
        _gate_body,
        grid=(_GRID,),
        in_specs=[
            pl.BlockSpec((_BB, _R, _D), lambda i: (i, 0, 0)),
            pl.BlockSpec((_D, _D), lambda i: (0, 0)),
            pl.BlockSpec((_D, 1), lambda i: (0, 0)),
            pl.BlockSpec((_D, _D), lambda i: (0, 0)),
            pl.BlockSpec((_D, 1), lambda i: (0, 0)),
            pl.BlockSpec((_NE, _D), lambda i: (0, 0)),
            pl.BlockSpec((_NE, 1), lambda i: (0, 0)),
        ],
        out_specs=[
            pl.BlockSpec((_NE, _BB), lambda i: (0, i)),
            pl.BlockSpec((_K, _BB), lambda i: (0, i)),
            pl.BlockSpec((_K, _BB), lambda i: (0, i)),
            pl.BlockSpec((1, _NE, 1), lambda i: (i, 0, 0)),
        ],
        out_shape=[
            jax.ShapeDtypeStruct((_NE, _B), jnp.float32),
            jax.ShapeDtypeStruct((_K, _B), jnp.float32),
            jax.ShapeDtypeStruct((_K, _B), jnp.int32),
            jax.ShapeDtypeStruct((_GRID, _NE, 1), jnp.float32),
        ],
        compiler_params=pltpu.CompilerParams(
            dimension_semantics=("parallel",),
        ),
    )(xt, W1, b1.reshape(_D, 1), W2, b2.reshape(_D, 1),
      W3, b3.reshape(_NE, 1))

    loss = pl.pallas_call(
        _loss_body,
        out_shape=jax.ShapeDtypeStruct((1, 1), jnp.float32),
    )(psum)

    return (probs_t.T, tkp_t.T, tki_t.T, loss.reshape(()))


# speedup vs baseline: 1.0603x; 1.0603x over previous
import jax
import jax.numpy as jnp
from jax.experimental import pallas as pl
from jax.experimental.pallas import tpu as pltpu

_NE = 64
_K = 8
_B = 4096
_BB = 512
_D = 128
_R = 32
_GRID = _B // _BB


def _gate_body(x_ref, w1_ref, b1_ref, w2_ref, b2_ref, w3_ref, b3_ref,
               probs_ref, tkp_ref, tki_ref, loss_ref, acc_ref):
    i = pl.program_id(0)

    xb = x_ref[...]
    y = jnp.sum(xb, axis=1) * (1.0 / _R)
    ht = jnp.transpose(y)
    h = jnp.maximum(jnp.dot(w1_ref[...], ht, preferred_element_type=jnp.float32)
                    + b1_ref[...], 0.0)
    h = jnp.maximum(jnp.dot(w2_ref[...], h, preferred_element_type=jnp.float32)
                    + b2_ref[...], 0.0)
    logits = (jnp.dot(w3_ref[...], h, preferred_element_type=jnp.float32)
              + b3_ref[...])

    m = jnp.max(logits, axis=0, keepdims=True)
    e = jnp.exp(logits - m)
    p = e / jnp.sum(e, axis=0, keepdims=True)
    probs_ref[...] = p

    @pl.when(i == 0)
    def _():
        acc_ref[...] = jnp.zeros_like(acc_ref)
    acc_ref[...] += p

    idx = jax.lax.broadcasted_iota(jnp.int32, p.shape, 0).astype(jnp.float32)
    pc = p
    for k in range(_K):
        mv = jnp.max(pc, axis=0, keepdims=True)
        sel = jnp.min(jnp.where(pc == mv, idx, float(_NE)),
                      axis=0, keepdims=True)
        tkp_ref[pl.ds(k, 1), :] = mv
        tki_ref[pl.ds(k, 1), :] = sel.astype(jnp.int32)
        pc = jnp.where(idx == sel, -1.0, pc)

    @pl.when(i == _GRID - 1)
    def _():
        colsum = jnp.sum(acc_ref[...], axis=1, keepdims=True)
        d = colsum * (1.0 / _B) - (1.0 / _NE)
        loss_ref[...] = jnp.sum(d * d, axis=0, keepdims=True)


def kernel(x, W1, b1, W2, b2, W3, b3):
    xt = jnp.transpose(x, (0, 1, 3, 4, 2)).reshape(_B, _R, _D)

    probs_t, tkp_t, tki_t, loss = pl.pallas_call(
        _gate_body,
        grid=(_GRID,),
        in_specs=[
            pl.BlockSpec((_BB, _R, _D), lambda i: (i, 0, 0)),
            pl.BlockSpec((_D, _D), lambda i: (0, 0)),
            pl.BlockSpec((_D, 1), lambda i: (0, 0)),
            pl.BlockSpec((_D, _D), lambda i: (0, 0)),
            pl.BlockSpec((_D, 1), lambda i: (0, 0)),
            pl.BlockSpec((_NE, _D), lambda i: (0, 0)),
            pl.BlockSpec((_NE, 1), lambda i: (0, 0)),
        ],
        out_specs=[
            pl.BlockSpec((_NE, _BB), lambda i: (0, i)),
            pl.BlockSpec((_K, _BB), lambda i: (0, i)),
            pl.BlockSpec((_K, _BB), lambda i: (0, i)),
            pl.BlockSpec((1, 1), lambda i: (0, 0)),
        ],
        out_shape=[
            jax.ShapeDtypeStruct((_NE, _B), jnp.float32),
            jax.ShapeDtypeStruct((_K, _B), jnp.float32),
            jax.ShapeDtypeStruct((_K, _B), jnp.int32),
            jax.ShapeDtypeStruct((1, 1), jnp.float32),
        ],
        scratch_shapes=[pltpu.VMEM((_NE, _BB), jnp.float32)],
        compiler_params=pltpu.CompilerParams(
            dimension_semantics=("arbitrary",),
        ),
    )(xt, W1, b1.reshape(_D, 1), W2, b2.reshape(_D, 1),
      W3, b3.reshape(_NE, 1))

    return (probs_t.T, tkp_t.T, tki_t.T, loss.reshape(()))
